# baseline (speedup 1.0000x reference)
import functools

import jax
import jax.numpy as jnp
import numpy as np
from jax.experimental import pallas as pl
from jax.experimental.pallas import tpu as pltpu

_NS = 32


def _fps_body(xs_ref, ys_ref, zs_ref, ox_ref, oy_ref, oz_ref, *, S, N, B):
    X = xs_ref[...]
    Y = ys_ref[...]
    Z = zs_ref[...]
    lane = jax.lax.broadcasted_iota(jnp.int32, (B, N), 1)
    lane_s = jax.lax.broadcasted_iota(jnp.int32, (B, S), 1)

    def step(t, carry):
        dist, cur = carry
        selm = lane == cur
        cx = jnp.sum(jnp.where(selm, X, 0.0), axis=1, keepdims=True)
        cy = jnp.sum(jnp.where(selm, Y, 0.0), axis=1, keepdims=True)
        cz = jnp.sum(jnp.where(selm, Z, 0.0), axis=1, keepdims=True)
        sel_t = lane_s == t
        ox_ref[...] = jnp.where(sel_t, cx, ox_ref[...])
        oy_ref[...] = jnp.where(sel_t, cy, oy_ref[...])
        oz_ref[...] = jnp.where(sel_t, cz, oz_ref[...])
        dx = X - cx
        dy = Y - cy
        dz = Z - cz
        d = dx * dx + dy * dy + dz * dz
        dist = jnp.minimum(dist, d)
        m = jnp.max(dist, axis=1, keepdims=True)
        nxt = jnp.min(jnp.where(dist == m, lane, N), axis=1, keepdims=True)
        return dist, nxt

    jax.lax.fori_loop(
        0, S, step,
        (jnp.full((B, N), 1e10, jnp.float32), jnp.zeros((B, 1), jnp.int32)))


def _fps_coords(xs, ys, zs, S):
    B, N = xs.shape
    outs = pl.pallas_call(
        functools.partial(_fps_body, S=S, N=N, B=B),
        in_specs=[pl.BlockSpec((B, N), lambda: (0, 0))] * 3,
        out_specs=[pl.BlockSpec((B, S), lambda: (0, 0))] * 3,
        out_shape=[jax.ShapeDtypeStruct((B, S), jnp.float32)] * 3,
    )(xs, ys, zs)
    return outs


_QB = 8


def _ball_body(pt_ref, q_ref, o_ref, *, r2, N):
    PT = pt_ref[0]
    Q = q_ref[0]
    lane = jax.lax.broadcasted_iota(jnp.int32, (_QB, N), 1)
    qx, qy, qz = Q[:, 0:1], Q[:, 1:2], Q[:, 2:3]
    qq = (qx * qx + qy * qy) + qz * qz
    X, Y, Z = PT[0:1], PT[1:2], PT[2:3]
    pp = (X * X + Y * Y) + Z * Z
    qp = jnp.dot(Q, PT, preferred_element_type=jnp.float32)
    d = (qq + pp) - 2.0 * qp
    hit = d <= r2
    rank = hit.astype(jnp.int32)
    k = 1
    while k < N:
        rank = rank + jnp.pad(rank, ((0, 0), (k, 0)))[:, :N]
        k *= 2
    g = jnp.where(hit, rank, 0)
    lanep1 = lane + 1
    vals = []
    for s in range(_NS):
        m = g == (s + 1)
        vals.append(jnp.sum(jnp.where(m, lanep1, 0), axis=1, keepdims=True) - 1)
    stacked = jnp.concatenate(vals, axis=1)
    stacked = jnp.where(stacked < 0, vals[0], stacked)
    o_ref[0] = stacked


def _ball_idx(pt, new_xyz, radius):
    B, _, N = pt.shape
    S = new_xyz.shape[1]
    out = pl.pallas_call(
        functools.partial(_ball_body, r2=radius * radius, N=N),
        grid=(B, S // _QB),
        in_specs=[pl.BlockSpec((1, 3, N), lambda b, i: (b, 0, 0)),
                  pl.BlockSpec((1, _QB, 3), lambda b, i: (b, i, 0))],
        out_specs=pl.BlockSpec((1, _QB, _NS), lambda b, i: (b, i, 0)),
        out_shape=jax.ShapeDtypeStruct((B, S, _NS), jnp.int32),
    )(pt, new_xyz)
    return out


def _sqdist(src, dst):
    return (jnp.sum(src ** 2, axis=-1)[:, :, None]
            + jnp.sum(dst ** 2, axis=-1)[:, None, :]
            - 2.0 * jnp.einsum('bnc,bmc->bnm', src, dst))


def _index_points(points, idx):
    B = points.shape[0]
    bidx = jnp.arange(B).reshape((B,) + (1,) * (idx.ndim - 1))
    return points[bidx, idx]


def _fps(xyz, npoint):
    xyz = jax.lax.stop_gradient(xyz)
    B, N, _ = xyz.shape

    def step(carry, _):
        distance, farthest = carry
        centroid = jnp.take_along_axis(xyz, farthest[:, None, None], axis=1)
        d = jnp.sum((xyz - centroid) ** 2, axis=-1)
        distance = jnp.minimum(distance, d)
        nxt = jnp.argmax(distance, axis=-1).astype(jnp.int32)
        return (distance, nxt), farthest

    init = (jnp.full((B, N), 1e10, dtype=xyz.dtype), jnp.zeros((B,), jnp.int32))
    _, cents = jax.lax.scan(step, init, None, length=npoint)
    return jnp.transpose(cents)


def _ball(radius, xyz, new_xyz):
    B, N, _ = xyz.shape
    S = new_xyz.shape[1]
    sqr = _sqdist(new_xyz, xyz)
    gi = jnp.broadcast_to(jnp.arange(N, dtype=jnp.int32), (B, S, N))
    gi = jnp.where(sqr > radius ** 2, N, gi)
    gi = jnp.sort(gi, axis=-1)[:, :, :_NS]
    first = gi[:, :, :1]
    gi = jnp.where(gi == N, jnp.broadcast_to(first, gi.shape), gi)
    return gi


def _fold(layers):
    out = []
    for p in layers:
        W = p['W'] * p['gamma'][None, :]
        b = p['b'] * p['gamma'] + p['beta']
        out.append((W, b[None, :]))
    return out


def _wspecs(folded):
    specs = []
    for (W, b) in folded:
        specs.append(pl.BlockSpec(W.shape, lambda *_: (0,) * W.ndim))
        specs.append(pl.BlockSpec(b.shape, lambda *_: (0,) * b.ndim))
    return specs


def _flatw(folded):
    out = []
    for (W, b) in folded:
        out.extend([W, b])
    return out


def _mm(a, w):
    return jnp.dot(a, w, preferred_element_type=jnp.float32)


def _sa_body(g_ref, w1, b1, w2, b2, w3, b3, o_ref, *, S):
    h = g_ref[0]
    h = jnp.maximum(_mm(h, w1[...]) + b1[...], 0.0)
    h = jnp.maximum(_mm(h, w2[...]) + b2[...], 0.0)
    h = jnp.maximum(_mm(h, w3[...]) + b3[...], 0.0)
    acc = h[0:S]
    for k in range(1, _NS):
        acc = jnp.maximum(acc, h[k * S:(k + 1) * S])
    o_ref[0] = acc


def _sa_mlp_pool(grouped, folded):
    B, NS, S, Cin = grouped.shape
    C3 = folded[-1][0].shape[1]
    g2 = grouped.reshape(B, NS * S, Cin)
    out = pl.pallas_call(
        functools.partial(_sa_body, S=S),
        grid=(B,),
        in_specs=[pl.BlockSpec((1, NS * S, Cin), lambda b: (b, 0, 0))] + _wspecs(folded),
        out_specs=pl.BlockSpec((1, S, C3), lambda b: (b, 0, 0)),
        out_shape=jax.ShapeDtypeStruct((B, S, C3), jnp.float32),
    )(g2, *_flatw(folded))
    return out


def _fp_body(p1_ref, it_ref, w1a, w1b, b1, w2, b2, o_ref):
    h = jnp.maximum(_mm(p1_ref[0], w1a[...]) + _mm(it_ref[0], w1b[...]) + b1[...], 0.0)
    h = jnp.maximum(_mm(h, w2[...]) + b2[...], 0.0)
    o_ref[0] = h


def _fp_mlp(points1, interp, folded):
    B, S, C1 = points1.shape
    C2 = interp.shape[2]
    (W1, b1), (W2, b2) = folded
    W1a, W1b = W1[:C1], W1[C1:]
    Cout = W2.shape[1]
    out = pl.pallas_call(
        _fp_body,
        grid=(B,),
        in_specs=[
            pl.BlockSpec((1, S, C1), lambda b: (b, 0, 0)),
            pl.BlockSpec((1, S, C2), lambda b: (b, 0, 0)),
            pl.BlockSpec(W1a.shape, lambda b: (0, 0)),
            pl.BlockSpec(W1b.shape, lambda b: (0, 0)),
            pl.BlockSpec(b1.shape, lambda b: (0, 0)),
            pl.BlockSpec(W2.shape, lambda b: (0, 0)),
            pl.BlockSpec(b2.shape, lambda b: (0, 0)),
        ],
        out_specs=pl.BlockSpec((1, S, Cout), lambda b: (b, 0, 0)),
        out_shape=jax.ShapeDtypeStruct((B, S, Cout), jnp.float32),
    )(points1, interp, W1a, W1b, b1[None] if b1.ndim == 1 else b1,
      W2, b2[None] if b2.ndim == 1 else b2)
    return out


def _fp1_body(it_ref, w1, b1, w2, b2, w3, b3, w4, b4, wc, bc, o_ref):
    h = it_ref[0]
    h = jnp.maximum(_mm(h, w1[...]) + b1[...], 0.0)
    h = jnp.maximum(_mm(h, w2[...]) + b2[...], 0.0)
    h = jnp.maximum(_mm(h, w3[...]) + b3[...], 0.0)
    h = jnp.maximum(_mm(h, w4[...]) + b4[...], 0.0)
    logits = _mm(h, wc[...]) + bc[...]
    m = jnp.max(logits, axis=-1, keepdims=True)
    e = logits - m
    lse = jnp.log(jnp.sum(jnp.exp(e), axis=-1, keepdims=True))
    o_ref[0] = e - lse


def _fp1_head(interp, folded, convW, convb):
    B, S, C = interp.shape
    NB = 8
    SB = S // NB
    NC = convW.shape[1]
    args = []
    for (W, b) in folded:
        args.extend([W, b])
    args.extend([convW, convb[None]])
    wsp = []
    for a in args:
        wsp.append(pl.BlockSpec(a.shape, lambda b, i: (0, 0)))
    out = pl.pallas_call(
        _fp1_body,
        grid=(B, NB),
        in_specs=[pl.BlockSpec((1, SB, C), lambda b, i: (b, i, 0))] + wsp,
        out_specs=pl.BlockSpec((1, SB, NC), lambda b, i: (b, i, 0)),
        out_shape=jax.ShapeDtypeStruct((B, S, NC), jnp.float32),
    )(interp, *args)
    return out


def _set_abstraction(xyz, points, npoint, radius, layers):
    xs, ys, zs = xyz[:, :, 0], xyz[:, :, 1], xyz[:, :, 2]
    qx, qy, qz = _fps_coords(xs, ys, zs, npoint)
    new_xyz = jnp.stack([qx, qy, qz], axis=-1)
    idx = _ball_idx(jnp.stack([xs, ys, zs], axis=1), new_xyz, radius)
    grouped_xyz = _index_points(xyz, idx) - new_xyz[:, :, None, :]
    if points is not None:
        grouped = jnp.concatenate([grouped_xyz, _index_points(points, idx)], axis=-1)
    else:
        grouped = grouped_xyz
    grouped = jnp.transpose(grouped, (0, 2, 1, 3))
    folded = _fold(layers)
    return new_xyz, _sa_mlp_pool(grouped, folded)


def _three_interp(xyz1, xyz2, points2):
    dists = _sqdist(xyz1, xyz2)
    negd, idx = jax.lax.top_k(-dists, 3)
    d = -negd
    w = 1.0 / (d + 1e-8)
    w = w / jnp.sum(w, axis=-1, keepdims=True)
    return jnp.sum(_index_points(points2, idx) * w[..., None], axis=2)


def _feature_propagation(xyz1, xyz2, points1, points2, layers):
    interp = _three_interp(xyz1, xyz2, points2)
    folded = _fold(layers)
    return _fp_mlp(points1, interp, folded)


def kernel(x, params):
    coords0 = x[:, :, :3]
    feats0 = x[:, :, 3:]
    c1, f1 = _set_abstraction(coords0, feats0, 1024, 0.1, params['sa1'])
    c2, f2 = _set_abstraction(c1, f1, 256, 0.2, params['sa2'])
    c3, f3 = _set_abstraction(c2, f2, 64, 0.4, params['sa3'])
    c4, f4 = _set_abstraction(c3, f3, 16, 0.8, params['sa4'])
    f3 = _feature_propagation(c3, c4, f3, f4, params['fp4'])
    f2 = _feature_propagation(c2, c3, f2, f3, params['fp3'])
    f1 = _feature_propagation(c1, c2, f1, f2, params['fp2'])
    interp0 = _three_interp(coords0, c1, f1)
    folded1 = _fold(params['fp1'])
    return _fp1_head(interp0, folded1, params['conv']['W'], params['conv']['b'])


# speedup vs baseline: 1.0132x; 1.0132x over previous
import functools

import jax
import jax.numpy as jnp
import numpy as np
from jax.experimental import pallas as pl
from jax.experimental.pallas import tpu as pltpu

_NS = 32


def _fps_body(xs_ref, ys_ref, zs_ref, ox_ref, oy_ref, oz_ref, *, S, N, B):
    X = xs_ref[...]
    Y = ys_ref[...]
    Z = zs_ref[...]
    lane = jax.lax.broadcasted_iota(jnp.int32, (B, N), 1)
    lane_s = jax.lax.broadcasted_iota(jnp.int32, (B, S), 1)

    def step(t, carry):
        dist, cur = carry
        selm = lane == cur
        cx = jnp.sum(jnp.where(selm, X, 0.0), axis=1, keepdims=True)
        cy = jnp.sum(jnp.where(selm, Y, 0.0), axis=1, keepdims=True)
        cz = jnp.sum(jnp.where(selm, Z, 0.0), axis=1, keepdims=True)
        sel_t = lane_s == t
        ox_ref[...] = jnp.where(sel_t, cx, ox_ref[...])
        oy_ref[...] = jnp.where(sel_t, cy, oy_ref[...])
        oz_ref[...] = jnp.where(sel_t, cz, oz_ref[...])
        dx = X - cx
        dy = Y - cy
        dz = Z - cz
        d = dx * dx + dy * dy + dz * dz
        dist = jnp.minimum(dist, d)
        m = jnp.max(dist, axis=1, keepdims=True)
        nxt = jnp.min(jnp.where(dist == m, lane, N), axis=1, keepdims=True)
        return dist, nxt

    jax.lax.fori_loop(
        0, S, step,
        (jnp.full((B, N), 1e10, jnp.float32), jnp.zeros((B, 1), jnp.int32)))


def _fps_coords(xs, ys, zs, S):
    B, N = xs.shape
    outs = pl.pallas_call(
        functools.partial(_fps_body, S=S, N=N, B=B),
        in_specs=[pl.BlockSpec((B, N), lambda: (0, 0))] * 3,
        out_specs=[pl.BlockSpec((B, S), lambda: (0, 0))] * 3,
        out_shape=[jax.ShapeDtypeStruct((B, S), jnp.float32)] * 3,
    )(xs, ys, zs)
    return outs


_QB = 8


def _ball_body(pt_ref, q_ref, o_ref, *, r2, N):
    PT = pt_ref[0]
    Q = q_ref[0]
    lane = jax.lax.broadcasted_iota(jnp.int32, (_QB, N), 1)
    qx, qy, qz = Q[:, 0:1], Q[:, 1:2], Q[:, 2:3]
    qq = (qx * qx + qy * qy) + qz * qz
    X, Y, Z = PT[0:1], PT[1:2], PT[2:3]
    pp = (X * X + Y * Y) + Z * Z
    qp = jnp.dot(Q, PT, preferred_element_type=jnp.float32)
    d = (qq + pp) - 2.0 * qp
    hit = d <= r2
    rank = hit.astype(jnp.int32)
    k = 1
    while k < N:
        rank = rank + jnp.pad(rank, ((0, 0), (k, 0)))[:, :N]
        k *= 2
    g = jnp.where(hit, rank, 0)
    lanep1 = lane + 1
    vals = []
    for s in range(_NS):
        m = g == (s + 1)
        vals.append(jnp.sum(jnp.where(m, lanep1, 0), axis=1, keepdims=True) - 1)
    stacked = jnp.concatenate(vals, axis=1)
    stacked = jnp.where(stacked < 0, vals[0], stacked)
    o_ref[0] = stacked


def _ball_idx(pt, new_xyz, radius):
    B, _, N = pt.shape
    S = new_xyz.shape[1]
    out = pl.pallas_call(
        functools.partial(_ball_body, r2=radius * radius, N=N),
        grid=(B, S // _QB),
        in_specs=[pl.BlockSpec((1, 3, N), lambda b, i: (b, 0, 0)),
                  pl.BlockSpec((1, _QB, 3), lambda b, i: (b, i, 0))],
        out_specs=pl.BlockSpec((1, _QB, _NS), lambda b, i: (b, i, 0)),
        out_shape=jax.ShapeDtypeStruct((B, S, _NS), jnp.int32),
    )(pt, new_xyz)
    return out


def _knn3_body(pt_ref, q_ref, oi_ref, ow_ref, *, N):
    PT = pt_ref[0]
    Q = q_ref[0]
    lane = jax.lax.broadcasted_iota(jnp.int32, (_QB, N), 1)
    qx, qy, qz = Q[:, 0:1], Q[:, 1:2], Q[:, 2:3]
    qq = (qx * qx + qy * qy) + qz * qz
    X, Y, Z = PT[0:1], PT[1:2], PT[2:3]
    pp = (X * X + Y * Y) + Z * Z
    qp = jnp.dot(Q, PT, preferred_element_type=jnp.float32)
    d = (qq + pp) - 2.0 * qp
    idxs = []
    ws = []
    for _ in range(3):
        m = jnp.min(d, axis=1, keepdims=True)
        i = jnp.min(jnp.where(d == m, lane, N), axis=1, keepdims=True)
        idxs.append(i)
        ws.append(1.0 / (m + 1e-8))
        d = jnp.where(lane == i, jnp.float32(3e38), d)
    wsum = (ws[0] + ws[1]) + ws[2]
    oi_ref[0] = jnp.concatenate(idxs, axis=1)
    ow_ref[0] = jnp.concatenate([w / wsum for w in ws], axis=1)


def _knn3(pt2, q):
    B, _, N = pt2.shape
    S = q.shape[1]
    oi, ow = pl.pallas_call(
        functools.partial(_knn3_body, N=N),
        grid=(B, S // _QB),
        in_specs=[pl.BlockSpec((1, 3, N), lambda b, i: (b, 0, 0)),
                  pl.BlockSpec((1, _QB, 3), lambda b, i: (b, i, 0))],
        out_specs=[pl.BlockSpec((1, _QB, 3), lambda b, i: (b, i, 0))] * 2,
        out_shape=[jax.ShapeDtypeStruct((B, S, 3), jnp.int32),
                   jax.ShapeDtypeStruct((B, S, 3), jnp.float32)],
    )(pt2, q)
    return oi, ow


def _sqdist(src, dst):
    return (jnp.sum(src ** 2, axis=-1)[:, :, None]
            + jnp.sum(dst ** 2, axis=-1)[:, None, :]
            - 2.0 * jnp.einsum('bnc,bmc->bnm', src, dst))


def _index_points(points, idx):
    B = points.shape[0]
    bidx = jnp.arange(B).reshape((B,) + (1,) * (idx.ndim - 1))
    return points[bidx, idx]


def _fps(xyz, npoint):
    xyz = jax.lax.stop_gradient(xyz)
    B, N, _ = xyz.shape

    def step(carry, _):
        distance, farthest = carry
        centroid = jnp.take_along_axis(xyz, farthest[:, None, None], axis=1)
        d = jnp.sum((xyz - centroid) ** 2, axis=-1)
        distance = jnp.minimum(distance, d)
        nxt = jnp.argmax(distance, axis=-1).astype(jnp.int32)
        return (distance, nxt), farthest

    init = (jnp.full((B, N), 1e10, dtype=xyz.dtype), jnp.zeros((B,), jnp.int32))
    _, cents = jax.lax.scan(step, init, None, length=npoint)
    return jnp.transpose(cents)


def _ball(radius, xyz, new_xyz):
    B, N, _ = xyz.shape
    S = new_xyz.shape[1]
    sqr = _sqdist(new_xyz, xyz)
    gi = jnp.broadcast_to(jnp.arange(N, dtype=jnp.int32), (B, S, N))
    gi = jnp.where(sqr > radius ** 2, N, gi)
    gi = jnp.sort(gi, axis=-1)[:, :, :_NS]
    first = gi[:, :, :1]
    gi = jnp.where(gi == N, jnp.broadcast_to(first, gi.shape), gi)
    return gi


def _fold(layers):
    out = []
    for p in layers:
        W = p['W'] * p['gamma'][None, :]
        b = p['b'] * p['gamma'] + p['beta']
        out.append((W, b[None, :]))
    return out


def _wspecs(folded):
    specs = []
    for (W, b) in folded:
        specs.append(pl.BlockSpec(W.shape, lambda *_: (0,) * W.ndim))
        specs.append(pl.BlockSpec(b.shape, lambda *_: (0,) * b.ndim))
    return specs


def _flatw(folded):
    out = []
    for (W, b) in folded:
        out.extend([W, b])
    return out


def _mm(a, w):
    return jnp.dot(a, w, preferred_element_type=jnp.float32)


def _sa_body(g_ref, w1, b1, w2, b2, w3, b3, o_ref, *, S):
    h = g_ref[0]
    h = jnp.maximum(_mm(h, w1[...]) + b1[...], 0.0)
    h = jnp.maximum(_mm(h, w2[...]) + b2[...], 0.0)
    h = jnp.maximum(_mm(h, w3[...]) + b3[...], 0.0)
    acc = h[0:S]
    for k in range(1, _NS):
        acc = jnp.maximum(acc, h[k * S:(k + 1) * S])
    o_ref[0] = acc


def _sa_mlp_pool(grouped, folded):
    B, NS, S, Cin = grouped.shape
    C3 = folded[-1][0].shape[1]
    g2 = grouped.reshape(B, NS * S, Cin)
    out = pl.pallas_call(
        functools.partial(_sa_body, S=S),
        grid=(B,),
        in_specs=[pl.BlockSpec((1, NS * S, Cin), lambda b: (b, 0, 0))] + _wspecs(folded),
        out_specs=pl.BlockSpec((1, S, C3), lambda b: (b, 0, 0)),
        out_shape=jax.ShapeDtypeStruct((B, S, C3), jnp.float32),
    )(g2, *_flatw(folded))
    return out


def _fp_body(p1_ref, it_ref, w1a, w1b, b1, w2, b2, o_ref):
    h = jnp.maximum(_mm(p1_ref[0], w1a[...]) + _mm(it_ref[0], w1b[...]) + b1[...], 0.0)
    h = jnp.maximum(_mm(h, w2[...]) + b2[...], 0.0)
    o_ref[0] = h


def _fp_mlp(points1, interp, folded):
    B, S, C1 = points1.shape
    C2 = interp.shape[2]
    (W1, b1), (W2, b2) = folded
    W1a, W1b = W1[:C1], W1[C1:]
    Cout = W2.shape[1]
    out = pl.pallas_call(
        _fp_body,
        grid=(B,),
        in_specs=[
            pl.BlockSpec((1, S, C1), lambda b: (b, 0, 0)),
            pl.BlockSpec((1, S, C2), lambda b: (b, 0, 0)),
            pl.BlockSpec(W1a.shape, lambda b: (0, 0)),
            pl.BlockSpec(W1b.shape, lambda b: (0, 0)),
            pl.BlockSpec(b1.shape, lambda b: (0, 0)),
            pl.BlockSpec(W2.shape, lambda b: (0, 0)),
            pl.BlockSpec(b2.shape, lambda b: (0, 0)),
        ],
        out_specs=pl.BlockSpec((1, S, Cout), lambda b: (b, 0, 0)),
        out_shape=jax.ShapeDtypeStruct((B, S, Cout), jnp.float32),
    )(points1, interp, W1a, W1b, b1[None] if b1.ndim == 1 else b1,
      W2, b2[None] if b2.ndim == 1 else b2)
    return out


def _fp1_body(it_ref, w1, b1, w2, b2, w3, b3, w4, b4, wc, bc, o_ref):
    h = it_ref[0]
    h = jnp.maximum(_mm(h, w1[...]) + b1[...], 0.0)
    h = jnp.maximum(_mm(h, w2[...]) + b2[...], 0.0)
    h = jnp.maximum(_mm(h, w3[...]) + b3[...], 0.0)
    h = jnp.maximum(_mm(h, w4[...]) + b4[...], 0.0)
    logits = _mm(h, wc[...]) + bc[...]
    m = jnp.max(logits, axis=-1, keepdims=True)
    e = logits - m
    lse = jnp.log(jnp.sum(jnp.exp(e), axis=-1, keepdims=True))
    o_ref[0] = e - lse


def _fp1_head(interp, folded, convW, convb):
    B, S, C = interp.shape
    NB = 8
    SB = S // NB
    NC = convW.shape[1]
    args = []
    for (W, b) in folded:
        args.extend([W, b])
    args.extend([convW, convb[None]])
    wsp = []
    for a in args:
        wsp.append(pl.BlockSpec(a.shape, lambda b, i: (0, 0)))
    out = pl.pallas_call(
        _fp1_body,
        grid=(B, NB),
        in_specs=[pl.BlockSpec((1, SB, C), lambda b, i: (b, i, 0))] + wsp,
        out_specs=pl.BlockSpec((1, SB, NC), lambda b, i: (b, i, 0)),
        out_shape=jax.ShapeDtypeStruct((B, S, NC), jnp.float32),
    )(interp, *args)
    return out


def _set_abstraction(xyz, points, npoint, radius, layers):
    xs, ys, zs = xyz[:, :, 0], xyz[:, :, 1], xyz[:, :, 2]
    qx, qy, qz = _fps_coords(xs, ys, zs, npoint)
    new_xyz = jnp.stack([qx, qy, qz], axis=-1)
    idx = _ball_idx(jnp.stack([xs, ys, zs], axis=1), new_xyz, radius)
    grouped_xyz = _index_points(xyz, idx) - new_xyz[:, :, None, :]
    if points is not None:
        grouped = jnp.concatenate([grouped_xyz, _index_points(points, idx)], axis=-1)
    else:
        grouped = grouped_xyz
    grouped = jnp.transpose(grouped, (0, 2, 1, 3))
    folded = _fold(layers)
    return new_xyz, _sa_mlp_pool(grouped, folded)


def _three_interp(xyz1, xyz2, points2):
    idx, w = _knn3(jnp.transpose(xyz2, (0, 2, 1)), xyz1)
    return jnp.sum(_index_points(points2, idx) * w[..., None], axis=2)


def _feature_propagation(xyz1, xyz2, points1, points2, layers):
    interp = _three_interp(xyz1, xyz2, points2)
    folded = _fold(layers)
    return _fp_mlp(points1, interp, folded)


def kernel(x, params):
    coords0 = x[:, :, :3]
    feats0 = x[:, :, 3:]
    c1, f1 = _set_abstraction(coords0, feats0, 1024, 0.1, params['sa1'])
    c2, f2 = _set_abstraction(c1, f1, 256, 0.2, params['sa2'])
    c3, f3 = _set_abstraction(c2, f2, 64, 0.4, params['sa3'])
    c4, f4 = _set_abstraction(c3, f3, 16, 0.8, params['sa4'])
    f3 = _feature_propagation(c3, c4, f3, f4, params['fp4'])
    f2 = _feature_propagation(c2, c3, f2, f3, params['fp3'])
    f1 = _feature_propagation(c1, c2, f1, f2, params['fp2'])
    interp0 = _three_interp(coords0, c1, f1)
    folded1 = _fold(params['fp1'])
    return _fp1_head(interp0, folded1, params['conv']['W'], params['conv']['b'])


# speedup vs baseline: 1.0158x; 1.0026x over previous
import functools

import jax
import jax.numpy as jnp
import numpy as np
from jax.experimental import pallas as pl
from jax.experimental.pallas import tpu as pltpu

_NS = 32


def _fps_body(xs_ref, ys_ref, zs_ref, ox_ref, oy_ref, oz_ref, oi_ref, *,
              S, N, B):
    X = xs_ref[...]
    Y = ys_ref[...]
    Z = zs_ref[...]
    lane = jax.lax.broadcasted_iota(jnp.int32, (B, N), 1)
    lane_s = jax.lax.broadcasted_iota(jnp.int32, (B, S), 1)

    def step(t, carry):
        dist, cur = carry
        selm = lane == cur
        cx = jnp.sum(jnp.where(selm, X, 0.0), axis=1, keepdims=True)
        cy = jnp.sum(jnp.where(selm, Y, 0.0), axis=1, keepdims=True)
        cz = jnp.sum(jnp.where(selm, Z, 0.0), axis=1, keepdims=True)
        sel_t = lane_s == t
        ox_ref[...] = jnp.where(sel_t, cx, ox_ref[...])
        oy_ref[...] = jnp.where(sel_t, cy, oy_ref[...])
        oz_ref[...] = jnp.where(sel_t, cz, oz_ref[...])
        oi_ref[...] = jnp.where(sel_t, cur, oi_ref[...])
        dx = X - cx
        dy = Y - cy
        dz = Z - cz
        d = dx * dx + dy * dy + dz * dz
        dist = jnp.minimum(dist, d)
        m = jnp.max(dist, axis=1, keepdims=True)
        nxt = jnp.min(jnp.where(dist == m, lane, N), axis=1, keepdims=True)
        return dist, nxt

    jax.lax.fori_loop(
        0, S, step,
        (jnp.full((B, N), 1e10, jnp.float32), jnp.zeros((B, 1), jnp.int32)))


def _fps_coords(xs, ys, zs, S):
    B, N = xs.shape
    outs = pl.pallas_call(
        functools.partial(_fps_body, S=S, N=N, B=B),
        in_specs=[pl.BlockSpec((B, N), lambda: (0, 0))] * 3,
        out_specs=[pl.BlockSpec((B, S), lambda: (0, 0))] * 4,
        out_shape=[jax.ShapeDtypeStruct((B, S), jnp.float32)] * 3
        + [jax.ShapeDtypeStruct((B, S), jnp.int32)],
    )(xs, ys, zs)
    return outs


_QB = 8


def _ball_body(pt_ref, q_ref, o_ref, *, r2, N):
    PT = pt_ref[0]
    Q = q_ref[0]
    lane = jax.lax.broadcasted_iota(jnp.int32, (_QB, N), 1)
    qx, qy, qz = Q[:, 0:1], Q[:, 1:2], Q[:, 2:3]
    qq = (qx * qx + qy * qy) + qz * qz
    X, Y, Z = PT[0:1], PT[1:2], PT[2:3]
    pp = (X * X + Y * Y) + Z * Z
    qp = jnp.dot(Q.astype(jnp.bfloat16), PT.astype(jnp.bfloat16),
                 preferred_element_type=jnp.float32)
    d = (qq + pp) - 2.0 * qp
    hit = d <= r2
    rank = hit.astype(jnp.int32)
    k = 1
    while k < N:
        rank = rank + jnp.pad(rank, ((0, 0), (k, 0)))[:, :N]
        k *= 2
    g = jnp.where(hit, rank, 0)
    lanep1 = lane + 1
    vals = []
    for s in range(_NS):
        m = g == (s + 1)
        vals.append(jnp.sum(jnp.where(m, lanep1, 0), axis=1, keepdims=True) - 1)
    stacked = jnp.concatenate(vals, axis=1)
    pad = jnp.where(vals[0] < 0, N, vals[0])
    stacked = jnp.where(stacked < 0, pad, stacked)
    o_ref[0] = stacked


def _ball_idx(pt, new_xyz, radius):
    B, _, N = pt.shape
    S = new_xyz.shape[1]
    out = pl.pallas_call(
        functools.partial(_ball_body, r2=radius * radius, N=N),
        grid=(B, S // _QB),
        in_specs=[pl.BlockSpec((1, 3, N), lambda b, i: (b, 0, 0)),
                  pl.BlockSpec((1, _QB, 3), lambda b, i: (b, i, 0))],
        out_specs=pl.BlockSpec((1, _QB, _NS), lambda b, i: (b, i, 0)),
        out_shape=jax.ShapeDtypeStruct((B, S, _NS), jnp.int32),
    )(pt, new_xyz)
    return out


def _knn3_body(pt_ref, q_ref, oi_ref, ow_ref, *, N):
    PT = pt_ref[0]
    Q = q_ref[0]
    lane = jax.lax.broadcasted_iota(jnp.int32, (_QB, N), 1)
    qx, qy, qz = Q[:, 0:1], Q[:, 1:2], Q[:, 2:3]
    qq = (qx * qx + qy * qy) + qz * qz
    X, Y, Z = PT[0:1], PT[1:2], PT[2:3]
    pp = (X * X + Y * Y) + Z * Z
    qp = jnp.dot(Q, PT, preferred_element_type=jnp.float32)
    d = (qq + pp) - 2.0 * qp
    idxs = []
    ws = []
    for _ in range(3):
        m = jnp.min(d, axis=1, keepdims=True)
        i = jnp.min(jnp.where(d == m, lane, N), axis=1, keepdims=True)
        idxs.append(i)
        ws.append(1.0 / (m + 1e-8))
        d = jnp.where(lane == i, jnp.float32(3e38), d)
    wsum = (ws[0] + ws[1]) + ws[2]
    oi_ref[0] = jnp.concatenate(idxs, axis=1)
    ow_ref[0] = jnp.concatenate([w / wsum for w in ws], axis=1)


def _knn3(pt2, q):
    B, _, N = pt2.shape
    S = q.shape[1]
    oi, ow = pl.pallas_call(
        functools.partial(_knn3_body, N=N),
        grid=(B, S // _QB),
        in_specs=[pl.BlockSpec((1, 3, N), lambda b, i: (b, 0, 0)),
                  pl.BlockSpec((1, _QB, 3), lambda b, i: (b, i, 0))],
        out_specs=[pl.BlockSpec((1, _QB, 3), lambda b, i: (b, i, 0))] * 2,
        out_shape=[jax.ShapeDtypeStruct((B, S, 3), jnp.int32),
                   jax.ShapeDtypeStruct((B, S, 3), jnp.float32)],
    )(pt2, q)
    return oi, ow


def _sqdist(src, dst):
    return (jnp.sum(src ** 2, axis=-1)[:, :, None]
            + jnp.sum(dst ** 2, axis=-1)[:, None, :]
            - 2.0 * jnp.einsum('bnc,bmc->bnm', src, dst))


def _index_points(points, idx):
    B = points.shape[0]
    bidx = jnp.arange(B).reshape((B,) + (1,) * (idx.ndim - 1))
    return points[bidx, idx]


def _fps(xyz, npoint):
    xyz = jax.lax.stop_gradient(xyz)
    B, N, _ = xyz.shape

    def step(carry, _):
        distance, farthest = carry
        centroid = jnp.take_along_axis(xyz, farthest[:, None, None], axis=1)
        d = jnp.sum((xyz - centroid) ** 2, axis=-1)
        distance = jnp.minimum(distance, d)
        nxt = jnp.argmax(distance, axis=-1).astype(jnp.int32)
        return (distance, nxt), farthest

    init = (jnp.full((B, N), 1e10, dtype=xyz.dtype), jnp.zeros((B,), jnp.int32))
    _, cents = jax.lax.scan(step, init, None, length=npoint)
    return jnp.transpose(cents)


def _ball(radius, xyz, new_xyz):
    B, N, _ = xyz.shape
    S = new_xyz.shape[1]
    sqr = _sqdist(new_xyz, xyz)
    gi = jnp.broadcast_to(jnp.arange(N, dtype=jnp.int32), (B, S, N))
    gi = jnp.where(sqr > radius ** 2, N, gi)
    gi = jnp.sort(gi, axis=-1)[:, :, :_NS]
    first = gi[:, :, :1]
    gi = jnp.where(gi == N, jnp.broadcast_to(first, gi.shape), gi)
    return gi


def _fold(layers):
    out = []
    for p in layers:
        out.append((p['W'], p['b'][None, :], p['gamma'][None, :],
                    p['beta'][None, :]))
    return out


def _wspecs(folded):
    specs = []
    for arrs in folded:
        for a in arrs:
            specs.append(pl.BlockSpec(a.shape, lambda *_: (0,) * a.ndim))
    return specs


def _flatw(folded):
    out = []
    for arrs in folded:
        out.extend(arrs)
    return out


def _mm(a, w):
    return jnp.dot(a.astype(jnp.bfloat16), w.astype(jnp.bfloat16),
                   preferred_element_type=jnp.float32)


def _layer(h, w, b, g, bt):
    h = _mm(h, w[...]) + b[...]
    h = h * g[...] + bt[...]
    return jnp.maximum(h, 0.0)


def _sa_body(g_ref, w1, b1, g1, t1, w2, b2, g2, t2, w3, b3, g3, t3,
             o_ref, *, S):
    h = g_ref[0]
    h = _layer(h, w1, b1, g1, t1)
    h = _layer(h, w2, b2, g2, t2)
    h = _layer(h, w3, b3, g3, t3)
    acc = h[0:S]
    for k in range(1, _NS):
        acc = jnp.maximum(acc, h[k * S:(k + 1) * S])
    o_ref[0] = acc


def _sa_mlp_pool(grouped, folded):
    B, NS, S, Cin = grouped.shape
    C3 = folded[-1][0].shape[1]
    g2 = grouped.reshape(B, NS * S, Cin)
    out = pl.pallas_call(
        functools.partial(_sa_body, S=S),
        grid=(B,),
        in_specs=[pl.BlockSpec((1, NS * S, Cin), lambda b: (b, 0, 0))] + _wspecs(folded),
        out_specs=pl.BlockSpec((1, S, C3), lambda b: (b, 0, 0)),
        out_shape=jax.ShapeDtypeStruct((B, S, C3), jnp.float32),
    )(g2, *_flatw(folded))
    return out


def _fp_body(p1_ref, it_ref, w1a, w1b, b1, g1, t1, w2, b2, g2, t2, o_ref):
    h = (_mm(p1_ref[0], w1a[...]) + _mm(it_ref[0], w1b[...])) + b1[...]
    h = jnp.maximum(h * g1[...] + t1[...], 0.0)
    h = _layer(h, w2, b2, g2, t2)
    o_ref[0] = h


def _fp_mlp(points1, interp, folded):
    B, S, C1 = points1.shape
    C2 = interp.shape[2]
    (W1, b1, g1, t1), (W2, b2, g2, t2) = folded
    W1a, W1b = W1[:C1], W1[C1:]
    Cout = W2.shape[1]
    args = [W1a, W1b, b1, g1, t1, W2, b2, g2, t2]
    out = pl.pallas_call(
        _fp_body,
        grid=(B,),
        in_specs=[
            pl.BlockSpec((1, S, C1), lambda b: (b, 0, 0)),
            pl.BlockSpec((1, S, C2), lambda b: (b, 0, 0)),
        ] + [pl.BlockSpec(a.shape, lambda b: (0, 0)) for a in args],
        out_specs=pl.BlockSpec((1, S, Cout), lambda b: (b, 0, 0)),
        out_shape=jax.ShapeDtypeStruct((B, S, Cout), jnp.float32),
    )(points1, interp, *args)
    return out


def _fp1_body(it_ref, w1, b1, g1, t1, w2, b2, g2, t2, w3, b3, g3, t3,
              w4, b4, g4, t4, wc, bc, o_ref):
    h = it_ref[0]
    h = _layer(h, w1, b1, g1, t1)
    h = _layer(h, w2, b2, g2, t2)
    h = _layer(h, w3, b3, g3, t3)
    h = _layer(h, w4, b4, g4, t4)
    logits = _mm(h, wc[...]) + bc[...]
    m = jnp.max(logits, axis=-1, keepdims=True)
    e = logits - m
    lse = jnp.log(jnp.sum(jnp.exp(e), axis=-1, keepdims=True))
    o_ref[0] = e - lse


def _fp1_head(interp, folded, convW, convb):
    B, S, C = interp.shape
    NB = 8
    SB = S // NB
    NC = convW.shape[1]
    args = []
    for arrs in folded:
        args.extend(arrs)
    args.extend([convW, convb[None]])
    wsp = []
    for a in args:
        wsp.append(pl.BlockSpec(a.shape, lambda b, i: (0, 0)))
    out = pl.pallas_call(
        _fp1_body,
        grid=(B, NB),
        in_specs=[pl.BlockSpec((1, SB, C), lambda b, i: (b, i, 0))] + wsp,
        out_specs=pl.BlockSpec((1, SB, NC), lambda b, i: (b, i, 0)),
        out_shape=jax.ShapeDtypeStruct((B, S, NC), jnp.float32),
    )(interp, *args)
    return out


def _set_abstraction(xyz, points, npoint, radius, layers):
    xs, ys, zs = xyz[:, :, 0], xyz[:, :, 1], xyz[:, :, 2]
    qx, qy, qz, _ = _fps_coords(xs, ys, zs, npoint)
    new_xyz = jnp.stack([qx, qy, qz], axis=-1)
    idx = _ball_idx(jnp.stack([xs, ys, zs], axis=1), new_xyz, radius)
    grouped_xyz = _index_points(xyz, idx) - new_xyz[:, :, None, :]
    if points is not None:
        grouped = jnp.concatenate([grouped_xyz, _index_points(points, idx)], axis=-1)
    else:
        grouped = grouped_xyz
    grouped = jnp.transpose(grouped, (0, 2, 1, 3))
    folded = _fold(layers)
    return new_xyz, _sa_mlp_pool(grouped, folded)


def _three_interp(xyz1, xyz2, points2):
    idx, w = _knn3(jnp.transpose(xyz2, (0, 2, 1)), xyz1)
    return jnp.sum(_index_points(points2, idx) * w[..., None], axis=2)


def _feature_propagation(xyz1, xyz2, points1, points2, layers):
    interp = _three_interp(xyz1, xyz2, points2)
    folded = _fold(layers)
    return _fp_mlp(points1, interp, folded)


def kernel(x, params):
    coords0 = x[:, :, :3]
    feats0 = x[:, :, 3:]
    c1, f1 = _set_abstraction(coords0, feats0, 1024, 0.1, params['sa1'])
    c2, f2 = _set_abstraction(c1, f1, 256, 0.2, params['sa2'])
    c3, f3 = _set_abstraction(c2, f2, 64, 0.4, params['sa3'])
    c4, f4 = _set_abstraction(c3, f3, 16, 0.8, params['sa4'])
    f3 = _feature_propagation(c3, c4, f3, f4, params['fp4'])
    f2 = _feature_propagation(c2, c3, f2, f3, params['fp3'])
    f1 = _feature_propagation(c1, c2, f1, f2, params['fp2'])
    interp0 = _three_interp(coords0, c1, f1)
    folded1 = _fold(params['fp1'])
    return _fp1_head(interp0, folded1, params['conv']['W'], params['conv']['b'])


# speedup vs baseline: 1.0159x; 1.0001x over previous
import functools

import jax
import jax.numpy as jnp
from jax.experimental import pallas as pl

_NS = 32


def _fps_body(xs_ref, ys_ref, zs_ref, ox_ref, oy_ref, oz_ref, oi_ref, *,
              S, N, B):
    X = xs_ref[...]
    Y = ys_ref[...]
    Z = zs_ref[...]
    lane = jax.lax.broadcasted_iota(jnp.int32, (B, N), 1)
    lane_s = jax.lax.broadcasted_iota(jnp.int32, (B, S), 1)

    def step(t, carry):
        dist, cur = carry
        selm = lane == cur
        cx = jnp.sum(jnp.where(selm, X, 0.0), axis=1, keepdims=True)
        cy = jnp.sum(jnp.where(selm, Y, 0.0), axis=1, keepdims=True)
        cz = jnp.sum(jnp.where(selm, Z, 0.0), axis=1, keepdims=True)
        sel_t = lane_s == t
        ox_ref[...] = jnp.where(sel_t, cx, ox_ref[...])
        oy_ref[...] = jnp.where(sel_t, cy, oy_ref[...])
        oz_ref[...] = jnp.where(sel_t, cz, oz_ref[...])
        oi_ref[...] = jnp.where(sel_t, cur, oi_ref[...])
        dx = X - cx
        dy = Y - cy
        dz = Z - cz
        d = dx * dx + dy * dy + dz * dz
        dist = jnp.minimum(dist, d)
        m = jnp.max(dist, axis=1, keepdims=True)
        nxt = jnp.min(jnp.where(dist == m, lane, N), axis=1, keepdims=True)
        return dist, nxt

    jax.lax.fori_loop(
        0, S, step,
        (jnp.full((B, N), 1e10, jnp.float32), jnp.zeros((B, 1), jnp.int32)))


def _fps_coords(xs, ys, zs, S):
    B, N = xs.shape
    outs = pl.pallas_call(
        functools.partial(_fps_body, S=S, N=N, B=B),
        in_specs=[pl.BlockSpec((B, N), lambda: (0, 0))] * 3,
        out_specs=[pl.BlockSpec((B, S), lambda: (0, 0))] * 4,
        out_shape=[jax.ShapeDtypeStruct((B, S), jnp.float32)] * 3
        + [jax.ShapeDtypeStruct((B, S), jnp.int32)],
    )(xs, ys, zs)
    return outs


_QB = 8


def _ball_body(pt_ref, q_ref, o_ref, *, r2, N):
    PT = pt_ref[0]
    Q = q_ref[0]
    lane = jax.lax.broadcasted_iota(jnp.int32, (_QB, N), 1)
    qx, qy, qz = Q[:, 0:1], Q[:, 1:2], Q[:, 2:3]
    qq = (qx * qx + qy * qy) + qz * qz
    X, Y, Z = PT[0:1], PT[1:2], PT[2:3]
    pp = (X * X + Y * Y) + Z * Z
    qp = jnp.dot(Q.astype(jnp.bfloat16), PT.astype(jnp.bfloat16),
                 preferred_element_type=jnp.float32)
    d = (qq + pp) - 2.0 * qp
    hit = d <= r2
    rank = hit.astype(jnp.int32)
    k = 1
    while k < N:
        rank = rank + jnp.pad(rank, ((0, 0), (k, 0)))[:, :N]
        k *= 2
    g = jnp.where(hit, rank, 0)
    lanep1 = lane + 1
    vals = []
    for s in range(_NS):
        m = g == (s + 1)
        vals.append(jnp.sum(jnp.where(m, lanep1, 0), axis=1, keepdims=True) - 1)
    stacked = jnp.concatenate(vals, axis=1)
    pad = jnp.where(vals[0] < 0, N, vals[0])
    stacked = jnp.where(stacked < 0, pad, stacked)
    o_ref[0] = stacked


def _ball_idx(pt, new_xyz, radius):
    B, _, N = pt.shape
    S = new_xyz.shape[1]
    out = pl.pallas_call(
        functools.partial(_ball_body, r2=radius * radius, N=N),
        grid=(B, S // _QB),
        in_specs=[pl.BlockSpec((1, 3, N), lambda b, i: (b, 0, 0)),
                  pl.BlockSpec((1, _QB, 3), lambda b, i: (b, i, 0))],
        out_specs=pl.BlockSpec((1, _QB, _NS), lambda b, i: (b, i, 0)),
        out_shape=jax.ShapeDtypeStruct((B, S, _NS), jnp.int32),
    )(pt, new_xyz)
    return out


def _knn3_body(pt_ref, q_ref, oi_ref, ow_ref, *, N):
    PT = pt_ref[0]
    Q = q_ref[0]
    lane = jax.lax.broadcasted_iota(jnp.int32, (_QB, N), 1)
    qx, qy, qz = Q[:, 0:1], Q[:, 1:2], Q[:, 2:3]
    qq = (qx * qx + qy * qy) + qz * qz
    X, Y, Z = PT[0:1], PT[1:2], PT[2:3]
    pp = (X * X + Y * Y) + Z * Z
    qp = jnp.dot(Q, PT, preferred_element_type=jnp.float32)
    d = (qq + pp) - 2.0 * qp
    idxs = []
    ws = []
    for _ in range(3):
        m = jnp.min(d, axis=1, keepdims=True)
        i = jnp.min(jnp.where(d == m, lane, N), axis=1, keepdims=True)
        idxs.append(i)
        ws.append(1.0 / (m + 1e-8))
        d = jnp.where(lane == i, jnp.float32(3e38), d)
    wsum = (ws[0] + ws[1]) + ws[2]
    oi_ref[0] = jnp.concatenate(idxs, axis=1)
    ow_ref[0] = jnp.concatenate([w / wsum for w in ws], axis=1)


def _knn3(pt2, q):
    B, _, N = pt2.shape
    S = q.shape[1]
    oi, ow = pl.pallas_call(
        functools.partial(_knn3_body, N=N),
        grid=(B, S // _QB),
        in_specs=[pl.BlockSpec((1, 3, N), lambda b, i: (b, 0, 0)),
                  pl.BlockSpec((1, _QB, 3), lambda b, i: (b, i, 0))],
        out_specs=[pl.BlockSpec((1, _QB, 3), lambda b, i: (b, i, 0))] * 2,
        out_shape=[jax.ShapeDtypeStruct((B, S, 3), jnp.int32),
                   jax.ShapeDtypeStruct((B, S, 3), jnp.float32)],
    )(pt2, q)
    return oi, ow


def _index_points(points, idx):
    B = points.shape[0]
    bidx = jnp.arange(B).reshape((B,) + (1,) * (idx.ndim - 1))
    return points[bidx, idx]


def _fold(layers):
    out = []
    for p in layers:
        out.append((p['W'], p['b'][None, :], p['gamma'][None, :],
                    p['beta'][None, :]))
    return out


def _wspecs(folded):
    specs = []
    for arrs in folded:
        for a in arrs:
            specs.append(pl.BlockSpec(a.shape, lambda *_: (0,) * a.ndim))
    return specs


def _flatw(folded):
    out = []
    for arrs in folded:
        out.extend(arrs)
    return out


def _mm(a, w):
    return jnp.dot(a.astype(jnp.bfloat16), w.astype(jnp.bfloat16),
                   preferred_element_type=jnp.float32)


def _layer(h, w, b, g, bt):
    h = _mm(h, w[...]) + b[...]
    h = h * g[...] + bt[...]
    return jnp.maximum(h, 0.0)


def _sa_body(g_ref, w1, b1, g1, t1, w2, b2, g2, t2, w3, b3, g3, t3,
             o_ref, *, S):
    h = g_ref[0]
    h = _layer(h, w1, b1, g1, t1)
    h = _layer(h, w2, b2, g2, t2)
    h = _layer(h, w3, b3, g3, t3)
    acc = h[0:S]
    for k in range(1, _NS):
        acc = jnp.maximum(acc, h[k * S:(k + 1) * S])
    o_ref[0] = acc


def _sa_mlp_pool(grouped, folded):
    B, NS, S, Cin = grouped.shape
    C3 = folded[-1][0].shape[1]
    g2 = grouped.reshape(B, NS * S, Cin)
    out = pl.pallas_call(
        functools.partial(_sa_body, S=S),
        grid=(B,),
        in_specs=[pl.BlockSpec((1, NS * S, Cin), lambda b: (b, 0, 0))] + _wspecs(folded),
        out_specs=pl.BlockSpec((1, S, C3), lambda b: (b, 0, 0)),
        out_shape=jax.ShapeDtypeStruct((B, S, C3), jnp.float32),
    )(g2, *_flatw(folded))
    return out


def _fp_body(p1_ref, it_ref, w1a, w1b, b1, g1, t1, w2, b2, g2, t2, o_ref):
    h = (_mm(p1_ref[0], w1a[...]) + _mm(it_ref[0], w1b[...])) + b1[...]
    h = jnp.maximum(h * g1[...] + t1[...], 0.0)
    h = _layer(h, w2, b2, g2, t2)
    o_ref[0] = h


def _fp_mlp(points1, interp, folded):
    B, S, C1 = points1.shape
    C2 = interp.shape[2]
    (W1, b1, g1, t1), (W2, b2, g2, t2) = folded
    W1a, W1b = W1[:C1], W1[C1:]
    Cout = W2.shape[1]
    args = [W1a, W1b, b1, g1, t1, W2, b2, g2, t2]
    out = pl.pallas_call(
        _fp_body,
        grid=(B,),
        in_specs=[
            pl.BlockSpec((1, S, C1), lambda b: (b, 0, 0)),
            pl.BlockSpec((1, S, C2), lambda b: (b, 0, 0)),
        ] + [pl.BlockSpec(a.shape, lambda b: (0, 0)) for a in args],
        out_specs=pl.BlockSpec((1, S, Cout), lambda b: (b, 0, 0)),
        out_shape=jax.ShapeDtypeStruct((B, S, Cout), jnp.float32),
    )(points1, interp, *args)
    return out


def _fp1_body(it_ref, w1, b1, g1, t1, w2, b2, g2, t2, w3, b3, g3, t3,
              w4, b4, g4, t4, wc, bc, o_ref):
    h = it_ref[0]
    h = _layer(h, w1, b1, g1, t1)
    h = _layer(h, w2, b2, g2, t2)
    h = _layer(h, w3, b3, g3, t3)
    h = _layer(h, w4, b4, g4, t4)
    logits = _mm(h, wc[...]) + bc[...]
    m = jnp.max(logits, axis=-1, keepdims=True)
    e = logits - m
    lse = jnp.log(jnp.sum(jnp.exp(e), axis=-1, keepdims=True))
    o_ref[0] = e - lse


def _fp1_head(interp, folded, convW, convb):
    B, S, C = interp.shape
    NB = 8
    SB = S // NB
    NC = convW.shape[1]
    args = []
    for arrs in folded:
        args.extend(arrs)
    args.extend([convW, convb[None]])
    wsp = []
    for a in args:
        wsp.append(pl.BlockSpec(a.shape, lambda b, i: (0, 0)))
    out = pl.pallas_call(
        _fp1_body,
        grid=(B, NB),
        in_specs=[pl.BlockSpec((1, SB, C), lambda b, i: (b, i, 0))] + wsp,
        out_specs=pl.BlockSpec((1, SB, NC), lambda b, i: (b, i, 0)),
        out_shape=jax.ShapeDtypeStruct((B, S, NC), jnp.float32),
    )(interp, *args)
    return out


def _set_abstraction(xyz, points, npoint, radius, layers):
    xs, ys, zs = xyz[:, :, 0], xyz[:, :, 1], xyz[:, :, 2]
    qx, qy, qz, _ = _fps_coords(xs, ys, zs, npoint)
    new_xyz = jnp.stack([qx, qy, qz], axis=-1)
    idx = _ball_idx(jnp.stack([xs, ys, zs], axis=1), new_xyz, radius)
    grouped_xyz = _index_points(xyz, idx) - new_xyz[:, :, None, :]
    if points is not None:
        grouped = jnp.concatenate([grouped_xyz, _index_points(points, idx)], axis=-1)
    else:
        grouped = grouped_xyz
    grouped = jnp.transpose(grouped, (0, 2, 1, 3))
    folded = _fold(layers)
    return new_xyz, _sa_mlp_pool(grouped, folded)


def _three_interp(xyz1, xyz2, points2):
    idx, w = _knn3(jnp.transpose(xyz2, (0, 2, 1)), xyz1)
    return jnp.sum(_index_points(points2, idx) * w[..., None], axis=2)


def _feature_propagation(xyz1, xyz2, points1, points2, layers):
    interp = _three_interp(xyz1, xyz2, points2)
    folded = _fold(layers)
    return _fp_mlp(points1, interp, folded)


def kernel(x, params):
    coords0 = x[:, :, :3]
    feats0 = x[:, :, 3:]
    c1, f1 = _set_abstraction(coords0, feats0, 1024, 0.1, params['sa1'])
    c2, f2 = _set_abstraction(c1, f1, 256, 0.2, params['sa2'])
    c3, f3 = _set_abstraction(c2, f2, 64, 0.4, params['sa3'])
    c4, f4 = _set_abstraction(c3, f3, 16, 0.8, params['sa4'])
    f3 = _feature_propagation(c3, c4, f3, f4, params['fp4'])
    f2 = _feature_propagation(c2, c3, f2, f3, params['fp3'])
    f1 = _feature_propagation(c1, c2, f1, f2, params['fp2'])
    interp0 = _three_interp(coords0, c1, f1)
    folded1 = _fold(params['fp1'])
    return _fp1_head(interp0, folded1, params['conv']['W'], params['conv']['b'])
